# baseline (speedup 1.0000x reference)
import functools
import jax
import jax.numpy as jnp
from jax import lax
from jax.experimental import pallas as pl
from jax.experimental.pallas import tpu as pltpu
from jax.experimental.pallas import tpu_sc as plsc

NI, NJ = 4096, 200
D = 64
B = NI * NJ
NC, NS = 2, 16
NW = NC * NS
IB = NI // NW
BPW = IB * NJ
NBUF = 4
L = 16

_mesh = plsc.VectorSubcoreMesh(core_axis_name="c", subcore_axis_name="s")


@functools.partial(
    pl.kernel,
    mesh=_mesh,
    out_type=jax.ShapeDtypeStruct((NJ, D // 8, NW, 8, IB), jnp.float32),
    scratch_types=[
        pltpu.VMEM((BPW,), jnp.int32),
        pltpu.VMEM((NJ, IB), jnp.int32),
        pltpu.VMEM((NBUF, IB, D), jnp.float32),
        pltpu.VMEM((NBUF, D, IB + 5), jnp.float32),
        pltpu.SemaphoreType.DMA((NBUF,)),
        pltpu.SemaphoreType.DMA((NBUF,)),
    ],
    compiler_params=pltpu.CompilerParams(
        use_tc_tiling_on_sc=False, needs_layout_passes=False
    ),
)
def _gather_kernel(table_hbm, idx_hbm, out_hbm, idx_raw, idx_t, rows_a,
                   tiles_b, gsem, wsem):
    wid = lax.axis_index("s") * NC + lax.axis_index("c")
    pltpu.sync_copy(idx_hbm.at[pl.ds(wid * BPW, BPW)], idx_raw)

    lanes = lax.iota(jnp.int32, L)
    bases = [lanes * NJ + (L * t) * NJ for t in range(IB // L)]

    def idx_tr(j, carry):
        for t in range(IB // L):
            v = plsc.load_gather(idx_raw, [bases[t] + j])
            idx_t[j, pl.ds(L * t, L)] = v
        return carry

    lax.fori_loop(0, NJ, idx_tr, 0, unroll=False)

    def gather(j, s):
        return pltpu.make_async_copy(
            table_hbm.at[idx_t.at[j]], rows_a.at[s], gsem.at[s])

    def writeback(j, s):
        copies = [
            pltpu.make_async_copy(
                tiles_b.at[s, pl.ds(8 * tk, 8), pl.ds(0, IB)],
                out_hbm.at[j, tk, wid],
                wsem.at[s],
            )
            for tk in range(D // 8)
        ]

        class _Group:
            def start(self):
                for c in copies:
                    c.start()

            def wait(self):
                for c in copies:
                    c.wait()

        return _Group()

    kvs = [lanes + L * t for t in range(D // L)]

    def transpose(s):
        def tr_body(i16, carry):
            for u in range(16):
                il = i16 * 16 + u
                ilv = jnp.broadcast_to(il, (L,))
                for t in range(D // L):
                    v = rows_a[s, il, pl.ds(L * t, L)]
                    plsc.store_scatter(tiles_b.at[s], [kvs[t], ilv], v)
            return carry

        lax.fori_loop(0, IB // 16, tr_body, 0)

    for s in range(NBUF):
        gather(s, s).start()

    def block(b, carry):
        for s in range(NBUF):
            j = b * NBUF + s
            gather(j, s).wait()
            transpose(s)
            writeback(j, s).start()

            @pl.when(j >= 2)
            def _retire():
                writeback(j - 2, (s + 2) % NBUF).wait()

            @pl.when(j + NBUF < NJ)
            def _prefetch():
                gather(j + NBUF, s).start()

        return carry

    lax.fori_loop(0, NJ // NBUF, block, 0)

    writeback(NJ - 2, 2).wait()
    writeback(NJ - 1, 3).wait()


def kernel(joint_indices, table):
    flat_idx = joint_indices.reshape(B).astype(jnp.int32)
    out = _gather_kernel(table, flat_idx)
    return jnp.transpose(out, (2, 4, 0, 1, 3)).reshape(NI, NJ, D)


# speedup vs baseline: 1.1065x; 1.1065x over previous
import functools
import jax
import jax.numpy as jnp
from jax import lax
from jax.experimental import pallas as pl
from jax.experimental.pallas import tpu as pltpu
from jax.experimental.pallas import tpu_sc as plsc

NI, NJ = 4096, 200
D = 64
B = NI * NJ
NC, NS = 2, 16
NW = NC * NS
IB = NI // NW
BPW = IB * NJ
NBUF = 4
L = 16

_mesh = plsc.VectorSubcoreMesh(core_axis_name="c", subcore_axis_name="s")


@functools.partial(
    pl.kernel,
    mesh=_mesh,
    out_type=jax.ShapeDtypeStruct((NJ, D // 8, NW, 8, IB), jnp.float32),
    scratch_types=[
        pltpu.VMEM((BPW,), jnp.int32),
        pltpu.VMEM((NJ, IB), jnp.int32),
        pltpu.VMEM((NBUF, IB, D), jnp.float32),
        pltpu.VMEM((NBUF, D, IB + 5), jnp.float32),
        pltpu.SemaphoreType.DMA((NBUF,)),
        pltpu.SemaphoreType.DMA((NBUF,)),
    ],
    compiler_params=pltpu.CompilerParams(
        use_tc_tiling_on_sc=False, needs_layout_passes=False
    ),
)
def _gather_kernel(table_hbm, idx_hbm, out_hbm, idx_raw, idx_t, rows_a,
                   tiles_b, gsem, wsem):
    wid = lax.axis_index("s") * NC + lax.axis_index("c")
    pltpu.sync_copy(idx_hbm.at[pl.ds(wid * BPW, BPW)], idx_raw)

    lanes = lax.iota(jnp.int32, L)
    bases = [lanes * NJ + (L * t) * NJ for t in range(IB // L)]

    def idx_tr(j, carry):
        for t in range(IB // L):
            v = plsc.load_gather(idx_raw, [bases[t] + j])
            idx_t[j, pl.ds(L * t, L)] = v
        return carry

    lax.fori_loop(0, NJ, idx_tr, 0, unroll=False)

    def gather(j, s):
        return pltpu.make_async_copy(
            table_hbm.at[idx_t.at[j]], rows_a.at[s], gsem.at[s])

    def writeback(j, s):
        copies = [
            pltpu.make_async_copy(
                tiles_b.at[s, pl.ds(8 * tk, 8), pl.ds(0, IB)],
                out_hbm.at[j, tk, wid],
                wsem.at[s],
            )
            for tk in range(D // 8)
        ]

        class _Group:
            def start(self):
                for c in copies:
                    c.start()

            def wait(self):
                for c in copies:
                    c.wait()

        return _Group()

    kvs = [lanes + L * t for t in range(D // L)]

    def transpose(s):
        def tr_body(i4, carry):
            for u in range(4):
                il = i4 * 4 + u
                ilv = jnp.broadcast_to(il, (L,))
                for t in range(D // L):
                    v = rows_a[s, il, pl.ds(L * t, L)]
                    plsc.store_scatter(tiles_b.at[s], [kvs[t], ilv], v)
            return carry

        lax.fori_loop(0, IB // 4, tr_body, 0)

    for s in range(NBUF):
        gather(s, s).start()

    def block(b, carry):
        for s in range(NBUF):
            j = b * NBUF + s
            gather(j, s).wait()
            transpose(s)
            writeback(j, s).start()

            @pl.when(j >= 2)
            def _retire():
                writeback(j - 2, (s + 2) % NBUF).wait()

            @pl.when(j + NBUF < NJ)
            def _prefetch():
                gather(j + NBUF, s).start()

        return carry

    lax.fori_loop(0, NJ // NBUF, block, 0)

    writeback(NJ - 2, 2).wait()
    writeback(NJ - 1, 3).wait()


def kernel(joint_indices, table):
    flat_idx = joint_indices.reshape(B).astype(jnp.int32)
    out = _gather_kernel(table, flat_idx)
    return jnp.transpose(out, (2, 4, 0, 1, 3)).reshape(NI, NJ, D)
